# baseline (speedup 1.0000x reference)
import math

import jax
import jax.numpy as jnp
from jax import lax
from jax.experimental import pallas as pl
from jax.experimental.pallas import tpu as pltpu
from jax.experimental.pallas import tpu_sc as plsc

_NC = 2
_NS = 16
_NW = _NC * _NS
_CHUNK = 64
_NBUF = 8
_PD = 128


def _make_body(CH, ROWS_W, PERIOD):
    def body(x_hbm, tok_hbm, pose_hbm, out_hbm, idx_v, rv, pos_sh,
             si, sg, sw):
        sid = lax.axis_index("s")
        wid = sid * _NC + lax.axis_index("c")
        @pl.when(sid == 0)
        def _():
            pltpu.sync_copy(pose_hbm, pos_sh)

        pltpu.sync_copy(x_hbm.at[wid], idx_v)
        plsc.subcore_barrier()
        base = wid * ROWS_W

        def init(c, b):
            pltpu.async_copy(pos_sh.at[lax.rem(c, PERIOD)], rv.at[b],
                             si.at[b])

        def init_wait(c, b):
            pltpu.make_async_copy(pos_sh.at[lax.rem(c, PERIOD)], rv.at[b],
                                  si.at[b]).wait()

        def gather(c, b):
            pltpu.async_copy(tok_hbm.at[idx_v.at[c]], rv.at[b], sg.at[b],
                             add=True)

        def gather_wait(c, b):
            pltpu.make_async_copy(
                tok_hbm.at[idx_v.at[c]], rv.at[b], sg.at[b]).wait()

        def write(c, b):
            pltpu.async_copy(
                rv.at[b], out_hbm.at[pl.ds(base + c * _CHUNK, _CHUNK)],
                sw.at[b])

        def write_wait(c, b):
            pltpu.make_async_copy(
                rv.at[b], out_hbm.at[pl.ds(base + c * _CHUNK, _CHUNK)],
                sw.at[b]).wait()

        for c0 in range(4):
            init(c0, c0)
        init_wait(0, 0)
        gather(0, 0)
        init_wait(1, 1)
        gather(1, 1)

        @pl.loop(0, CH, step=_NBUF)
        def _(t):
            for k in range(_NBUF):
                c = t + k
                b2 = (k + 2) % _NBUF
                b4 = (k + 4) % _NBUF

                @pl.when(c + 4 < CH)
                def _():
                    @pl.when(c >= _NBUF - 4)
                    def _():
                        write_wait(c + 4 - _NBUF, b4)

                    init(c + 4, b4)

                @pl.when(c + 2 < CH)
                def _():
                    init_wait(c + 2, b2)
                    gather(c + 2, b2)

                gather_wait(c, k)
                write(c, k)

        for k in range(_NBUF):
            write_wait(CH - _NBUF + k, k)

    return body


def _slice_body(in_ref, out_ref):
    sb, l, d = out_ref.shape
    out_ref[...] = in_ref[...][:, :d].reshape(sb, l, d)


def kernel(x, token_table, pos_table):
    B, L = x.shape
    V, D = token_table.shape
    N = B * L
    ROWS_W = N // _NW
    CH = ROWS_W // _CHUNK
    PERIOD = math.lcm(_CHUNK, L) // _CHUNK

    x_r = x.reshape(_NW, CH, _CHUNK)
    tok_pad = jnp.pad(token_table, ((0, 0), (0, _PD - D)))
    reps = PERIOD * _CHUNK // L
    pos_exp = jnp.pad(
        jnp.tile(pos_table, (reps, 1)).reshape(PERIOD, _CHUNK, D),
        ((0, 0), (0, 0), (0, _PD - D)))

    mesh = plsc.VectorSubcoreMesh(core_axis_name="c", subcore_axis_name="s")
    out = pl.kernel(
        _make_body(CH, ROWS_W, PERIOD),
        out_type=jax.ShapeDtypeStruct((N, _PD), jnp.float32),
        mesh=mesh,
        scratch_types=[
            pltpu.VMEM((CH, _CHUNK), jnp.int32),
            pltpu.VMEM((_NBUF, _CHUNK, _PD), jnp.float32),
            pltpu.VMEM_SHARED((25, _CHUNK, _PD), jnp.float32),
            pltpu.SemaphoreType.DMA((_NBUF,)),
            pltpu.SemaphoreType.DMA((_NBUF,)),
            pltpu.SemaphoreType.DMA((_NBUF,)),
        ],
    )(x_r, tok_pad, pos_exp)

    SB = 64
    return pl.pallas_call(
        _slice_body,
        grid=(B // SB,),
        in_specs=[pl.BlockSpec((SB * L, _PD), lambda i: (i, 0))],
        out_specs=pl.BlockSpec((SB, L, D), lambda i: (i, 0, 0)),
        out_shape=jax.ShapeDtypeStruct((B, L, D), jnp.float32),
        compiler_params=pltpu.CompilerParams(
            dimension_semantics=("parallel",)),
    )(out)


# speedup vs baseline: 1.6766x; 1.6766x over previous
import math

import jax
import jax.numpy as jnp
from jax import lax
from jax.experimental import pallas as pl
from jax.experimental.pallas import tpu as pltpu
from jax.experimental.pallas import tpu_sc as plsc

_NC = 2
_NS = 16
_NW = _NC * _NS
_CHUNK = 64
_NBUF = 8
_PD = 128


def _make_body(CH, ROWS_W, PERIOD):
    def body(x_hbm, tok_hbm, pose_hbm, out_hbm, idx_v, rv, pos_sh,
             si, sg, sw):
        sid = lax.axis_index("s")
        wid = sid * _NC + lax.axis_index("c")
        @pl.when(sid == 0)
        def _():
            pltpu.sync_copy(pose_hbm, pos_sh)

        pltpu.sync_copy(x_hbm.at[wid], idx_v)
        plsc.subcore_barrier()
        base = wid * ROWS_W

        def init(c, b):
            pltpu.async_copy(pos_sh.at[lax.rem(c, PERIOD)], rv.at[b],
                             si.at[b])

        def init_wait(c, b):
            pltpu.make_async_copy(pos_sh.at[lax.rem(c, PERIOD)], rv.at[b],
                                  si.at[b]).wait()

        def gather(c, b):
            pltpu.async_copy(tok_hbm.at[idx_v.at[c]], rv.at[b], sg.at[b],
                             add=True)

        def gather_wait(c, b):
            pltpu.make_async_copy(
                tok_hbm.at[idx_v.at[c]], rv.at[b], sg.at[b]).wait()

        def write(c, b):
            pltpu.async_copy(
                rv.at[b], out_hbm.at[pl.ds(base + c * _CHUNK, _CHUNK)],
                sw.at[b])

        def write_wait(c, b):
            pltpu.make_async_copy(
                rv.at[b], out_hbm.at[pl.ds(base + c * _CHUNK, _CHUNK)],
                sw.at[b]).wait()

        for c0 in range(4):
            init(c0, c0)
        init_wait(0, 0)
        gather(0, 0)
        init_wait(1, 1)
        gather(1, 1)

        @pl.loop(0, CH, step=_NBUF)
        def _(t):
            for k in range(_NBUF):
                c = t + k
                b2 = (k + 2) % _NBUF
                b4 = (k + 4) % _NBUF

                @pl.when(c + 4 < CH)
                def _():
                    @pl.when(c >= _NBUF - 4)
                    def _():
                        write_wait(c + 4 - _NBUF, b4)

                    init(c + 4, b4)

                @pl.when(c + 2 < CH)
                def _():
                    init_wait(c + 2, b2)
                    gather(c + 2, b2)

                gather_wait(c, k)
                write(c, k)

        for k in range(_NBUF):
            write_wait(CH - _NBUF + k, k)

    return body


def kernel(x, token_table, pos_table):
    B, L = x.shape
    V, D = token_table.shape
    N = B * L
    ROWS_W = N // _NW
    CH = ROWS_W // _CHUNK
    PERIOD = math.lcm(_CHUNK, L) // _CHUNK

    x_r = x.reshape(_NW, CH, _CHUNK)
    tok_pad = jnp.pad(token_table, ((0, 0), (0, _PD - D)))
    reps = PERIOD * _CHUNK // L
    pos_exp = jnp.pad(
        jnp.tile(pos_table, (reps, 1)).reshape(PERIOD, _CHUNK, D),
        ((0, 0), (0, 0), (0, _PD - D)))

    mesh = plsc.VectorSubcoreMesh(core_axis_name="c", subcore_axis_name="s")
    out = pl.kernel(
        _make_body(CH, ROWS_W, PERIOD),
        out_type=jax.ShapeDtypeStruct((N, _PD), jnp.float32),
        mesh=mesh,
        scratch_types=[
            pltpu.VMEM((CH, _CHUNK), jnp.int32),
            pltpu.VMEM((_NBUF, _CHUNK, _PD), jnp.float32),
            pltpu.VMEM_SHARED((25, _CHUNK, _PD), jnp.float32),
            pltpu.SemaphoreType.DMA((_NBUF,)),
            pltpu.SemaphoreType.DMA((_NBUF,)),
            pltpu.SemaphoreType.DMA((_NBUF,)),
        ],
    )(x_r, tok_pad, pos_exp)
    return out[:, :D].reshape(B, L, D)
